# baseline (speedup 1.0000x reference)
import functools

import jax
import jax.numpy as jnp
from jax import lax
from jax.experimental import pallas as pl
from jax.experimental.pallas import tpu as pltpu
from jax.experimental.pallas import tpu_sc as plsc

D_MODEL = 128
HALF = 64
N_NUM = 8

NC = 2
NS = 16
NW = NC * NS

GATHER_CHUNK = 512


def _sc_gather_fn(n_tokens):
    b_per_w = n_tokens // NW
    n_chunks = b_per_w // GATHER_CHUNK

    mesh = plsc.VectorSubcoreMesh(core_axis_name="c", subcore_axis_name="s")

    @functools.partial(
        pl.kernel,
        mesh=mesh,
        out_type=jax.ShapeDtypeStruct((n_tokens, HALF), jnp.float32),
        scratch_types=[
            pltpu.VMEM((GATHER_CHUNK,), jnp.int32),
            pltpu.VMEM((GATHER_CHUNK, HALF), jnp.float32),
            pltpu.SemaphoreType.DMA,
        ],
        compiler_params=pltpu.CompilerParams(use_tc_tiling_on_sc=False),
    )
    def gather_k(table_hbm, idx_hbm, out_hbm, idx_v, rows_v, sem):
        wid = lax.axis_index("s") * NC + lax.axis_index("c")
        base = wid * b_per_w

        def body(i, carry):
            off = pl.multiple_of(base + i * GATHER_CHUNK, GATHER_CHUNK)
            pltpu.sync_copy(idx_hbm.at[pl.ds(off, GATHER_CHUNK)], idx_v)
            pltpu.async_copy(table_hbm.at[idx_v], rows_v, sem).wait()
            pltpu.sync_copy(rows_v, out_hbm.at[pl.ds(off, GATHER_CHUNK)])
            return carry

        lax.fori_loop(0, n_chunks, body, 0, unroll=False)

    return gather_k


def _tail_body(g_ref, nf_ref, wn_ref, bn_ref, wo_ref, bo_ref, gm_ref,
               bt_ref, o_ref):
    wo = wo_ref[...]
    wt = wo[:HALF]
    wo_b = wo[HALF:]
    wc = jnp.dot(wn_ref[...], wo_b, preferred_element_type=jnp.float32,
                 precision=lax.Precision.HIGHEST)
    bc = jnp.dot(bn_ref[...], wo_b, preferred_element_type=jnp.float32,
                 precision=lax.Precision.HIGHEST) + bo_ref[...]
    contrib = jnp.dot(nf_ref[...], wc, preferred_element_type=jnp.float32,
                      precision=lax.Precision.HIGHEST)
    t2 = g_ref.shape[0]
    c3 = contrib.reshape(t2, 2, D_MODEL)
    g2 = g_ref[...]
    gm = gm_ref[...]
    bt = bt_ref[...]

    def half(gpart, cpart):
        out = jnp.dot(gpart, wt, preferred_element_type=jnp.float32,
                      precision=lax.Precision.HIGHEST) + cpart + bc
        mean = jnp.mean(out, axis=-1, keepdims=True)
        cent = out - mean
        var = jnp.mean(cent * cent, axis=-1, keepdims=True)
        return cent * lax.rsqrt(var + 1e-5) * gm + bt

    o_ref[:, 0, :] = half(g2[:, :HALF], c3[:, 0, :])
    o_ref[:, 1, :] = half(g2[:, HALF:], c3[:, 1, :])


def _tc_tail(gathered2, nf, W_num, b_num, W_out, b_out, gamma, beta,
             pair_blk=2048):
    n_pairs = gathered2.shape[0]
    return pl.pallas_call(
        _tail_body,
        grid=(n_pairs // pair_blk,),
        in_specs=[
            pl.BlockSpec((pair_blk, D_MODEL), lambda i: (i, 0)),
            pl.BlockSpec((2 * pair_blk, N_NUM), lambda i: (i, 0)),
            pl.BlockSpec((N_NUM, HALF), lambda i: (0, 0)),
            pl.BlockSpec((1, HALF), lambda i: (0, 0)),
            pl.BlockSpec((D_MODEL, D_MODEL), lambda i: (0, 0)),
            pl.BlockSpec((1, D_MODEL), lambda i: (0, 0)),
            pl.BlockSpec((1, D_MODEL), lambda i: (0, 0)),
            pl.BlockSpec((1, D_MODEL), lambda i: (0, 0)),
        ],
        out_specs=pl.BlockSpec((pair_blk, 2, D_MODEL), lambda i: (i, 0, 0)),
        out_shape=jax.ShapeDtypeStruct((n_pairs, 2, D_MODEL), jnp.float32),
    )(gathered2, nf, W_num, b_num, W_out, b_out, gamma, beta)


def kernel(event_types, numerical_features, event_table, W_num, b_num,
           W_out, b_out, gamma, beta):
    B, L = event_types.shape
    n_tokens = B * L
    idx = event_types.reshape(n_tokens).astype(jnp.int32)
    gathered = _sc_gather_fn(n_tokens)(event_table, idx)
    gathered2 = gathered.reshape(n_tokens // 2, D_MODEL)
    nf = numerical_features.reshape(n_tokens, N_NUM)
    out = _tc_tail(gathered2, nf, W_num, b_num.reshape(1, HALF), W_out,
                   b_out.reshape(1, D_MODEL), gamma.reshape(1, D_MODEL),
                   beta.reshape(1, D_MODEL))
    return out.reshape(B, L, D_MODEL)
